# Initial kernel scaffold; baseline (speedup 1.0000x reference)
import jax
import jax.numpy as jnp
from jax.experimental import pallas as pl


def kernel(X, train_points):
    raise NotImplementedError("write your pallas kernel here")


# speedup vs baseline: 1.9393x; 1.9393x over previous
import jax
import jax.numpy as jnp
from jax.experimental import pallas as pl
from jax.experimental.pallas import tpu as pltpu

_K = 8
_LAMBDA = 3.0
_N = 100000
_NP = 102400
_D = 16
_Q = 8
_PAD = 1.0e6
_MASKV = 1.0e30


def _topk8(d2, iota, iota8):
    def body(j, carry):
        d2c, s, inds = carry
        m = jnp.min(d2c, axis=1, keepdims=True)
        cand = jnp.where(d2c == m, iota, _NP)
        idx = jnp.min(cand, axis=1, keepdims=True)
        d2c = jnp.where(iota == idx, _MASKV, d2c)
        s = s + m
        inds = jnp.where(iota8 == j, jnp.broadcast_to(idx, (_Q, _K)), inds)
        return (d2c, s, inds)

    init = (d2, jnp.zeros((_Q, 1), jnp.float32), jnp.zeros((_Q, _K), jnp.int32))
    _, s, inds = jax.lax.fori_loop(0, _K, body, init)
    return s, inds


def _pdist_from_sum(s):
    s = jnp.maximum(s, 0.0)
    return _LAMBDA * jnp.sqrt(s * (1.0 / _K))


def _erf_pos(x):
    t = 1.0 / (1.0 + 0.3275911 * x)
    poly = ((((1.061405429 * t - 1.453152027) * t + 1.421413741) * t
             - 0.284496736) * t + 0.254829592) * t
    return 1.0 - poly * jnp.exp(-(x * x))


def _loop_kernel(x_ref, tt_ref, out_ref):
    tt = tt_ref[:]
    tn = jnp.sum(tt * tt, axis=0, keepdims=True)
    iota = jax.lax.broadcasted_iota(jnp.int32, (_Q, _NP), 1)
    iota8 = jax.lax.broadcasted_iota(jnp.int32, (_Q, _K), 1)

    x = x_ref[:]
    qt = jax.lax.dot_general(x, tt, (((1,), (0,)), ((), ())),
                             preferred_element_type=jnp.float32,
                             precision=jax.lax.Precision.HIGHEST)
    qn = jnp.sum(x * x, axis=1, keepdims=True)
    d2 = tn - 2.0 * qt + qn
    s1, inds1 = _topk8(d2, iota, iota8)
    pd = _pdist_from_sum(s1)

    eye = (jax.lax.broadcasted_iota(jnp.int32, (_Q, _Q), 0)
           == jax.lax.broadcasted_iota(jnp.int32, (_Q, _Q), 1)).astype(jnp.float32)

    def group_body(g, nf_sum):
        col = jnp.min(jnp.where(iota8 == g, inds1, _NP), axis=1, keepdims=True)
        onehot = (iota == col).astype(jnp.float32)
        q2t = jax.lax.dot_general(tt, onehot, (((1,), (1,)), ((), ())),
                                  preferred_element_type=jnp.float32,
                             precision=jax.lax.Precision.HIGHEST)
        qt2 = jax.lax.dot_general(q2t, tt, (((0,), (0,)), ((), ())),
                                  preferred_element_type=jnp.float32,
                             precision=jax.lax.Precision.HIGHEST)
        gram = jax.lax.dot_general(q2t, q2t, (((0,), (0,)), ((), ())),
                                   preferred_element_type=jnp.float32,
                             precision=jax.lax.Precision.HIGHEST)
        qn2 = jnp.sum(gram * eye, axis=1, keepdims=True)
        d2b = tn - 2.0 * qt2 + qn2
        s2, _ = _topk8(d2b, iota, iota8)
        return nf_sum + _pdist_from_sum(s2)

    nf_sum = jax.lax.fori_loop(0, _K, group_body,
                               jnp.zeros((_Q, 1), jnp.float32))

    nf = nf_sum * (1.0 / _K)
    lof = pd / nf - 1.0
    y = lof * 0.7071067811865476
    res = jnp.where(y > 0.0, _erf_pos(jnp.maximum(y, 0.0)), 0.0)
    out_ref[:] = res


@jax.jit
def _run(x, train_points):
    pad = jnp.full((_NP - _N, _D), _PAD, dtype=jnp.float32)
    tt = jnp.concatenate([train_points, pad], axis=0).T
    out = pl.pallas_call(
        _loop_kernel,
        out_shape=jax.ShapeDtypeStruct((_Q, 1), jnp.float32),
        compiler_params=pltpu.CompilerParams(
            vmem_limit_bytes=100 * 1024 * 1024),
    )(x, tt)
    return out.reshape((_Q,))


def kernel(X, train_points):
    return _run(X, train_points)


# speedup vs baseline: 2.1417x; 1.1044x over previous
import jax
import jax.numpy as jnp
from jax.experimental import pallas as pl
from jax.experimental.pallas import tpu as pltpu

_K = 8
_LAMBDA = 3.0
_N = 100000
_NP = 102400
_D = 16
_Q = 8
_PAD = 1.0e6
_MASKV = 1.0e30


def _min_lanes(a):
    w = a.shape[1]
    while w > 3200:
        w //= 2
        a = jnp.minimum(a[:, :w], a[:, w:])
    return jnp.min(a, axis=1, keepdims=True)


def _topk8(d2, iota, iota8):
    def body(j, carry):
        d2c, s, inds = carry
        m = _min_lanes(d2c)
        cand = jnp.where(d2c == m, iota, _NP)
        idx = _min_lanes(cand)
        d2c = jnp.where(iota == idx, _MASKV, d2c)
        s = s + m
        inds = jnp.where(iota8 == j, jnp.broadcast_to(idx, (_Q, _K)), inds)
        return (d2c, s, inds)

    init = (d2, jnp.zeros((_Q, 1), jnp.float32), jnp.zeros((_Q, _K), jnp.int32))
    _, s, inds = jax.lax.fori_loop(0, _K, body, init)
    return s, inds


def _pdist_from_sum(s):
    s = jnp.maximum(s, 0.0)
    return _LAMBDA * jnp.sqrt(s * (1.0 / _K))


def _erf_pos(x):
    t = 1.0 / (1.0 + 0.3275911 * x)
    poly = ((((1.061405429 * t - 1.453152027) * t + 1.421413741) * t
             - 0.284496736) * t + 0.254829592) * t
    return 1.0 - poly * jnp.exp(-(x * x))


def _loop_kernel(x_ref, tt_ref, tt_hi_ref, tt_mid_ref, tt_lo_ref, out_ref):
    tt = tt_ref[:]
    tn = jnp.sum(tt * tt, axis=0, keepdims=True)
    iota = jax.lax.broadcasted_iota(jnp.int32, (_Q, _NP), 1)
    iota8 = jax.lax.broadcasted_iota(jnp.int32, (_Q, _K), 1)

    x = x_ref[:]
    qt = jax.lax.dot_general(x, tt, (((1,), (0,)), ((), ())),
                             preferred_element_type=jnp.float32,
                             precision=jax.lax.Precision.HIGHEST)
    qn = jnp.sum(x * x, axis=1, keepdims=True)
    d2 = tn - 2.0 * qt + qn
    s1, inds1 = _topk8(d2, iota, iota8)
    pd = _pdist_from_sum(s1)

    eye = (jax.lax.broadcasted_iota(jnp.int32, (_Q, _Q), 0)
           == jax.lax.broadcasted_iota(jnp.int32, (_Q, _Q), 1)).astype(jnp.float32)

    def group_body(g, nf_sum):
        col = jnp.min(jnp.where(iota8 == g, inds1, _NP), axis=1, keepdims=True)
        onehot = (iota == col).astype(jnp.bfloat16)
        q2t = (jax.lax.dot_general(tt_hi_ref[:], onehot, (((1,), (1,)), ((), ())),
                                   preferred_element_type=jnp.float32)
               + jax.lax.dot_general(tt_mid_ref[:], onehot, (((1,), (1,)), ((), ())),
                                     preferred_element_type=jnp.float32)
               + jax.lax.dot_general(tt_lo_ref[:], onehot, (((1,), (1,)), ((), ())),
                                     preferred_element_type=jnp.float32))
        qt2 = jax.lax.dot_general(q2t, tt, (((0,), (0,)), ((), ())),
                                  preferred_element_type=jnp.float32,
                             precision=jax.lax.Precision.HIGHEST)
        gram = jax.lax.dot_general(q2t, q2t, (((0,), (0,)), ((), ())),
                                   preferred_element_type=jnp.float32,
                             precision=jax.lax.Precision.HIGHEST)
        qn2 = jnp.sum(gram * eye, axis=1, keepdims=True)
        d2b = tn - 2.0 * qt2 + qn2
        s2, _ = _topk8(d2b, iota, iota8)
        return nf_sum + _pdist_from_sum(s2)

    nf_sum = jax.lax.fori_loop(0, _K, group_body,
                               jnp.zeros((_Q, 1), jnp.float32))

    nf = nf_sum * (1.0 / _K)
    lof = pd / nf - 1.0
    y = lof * 0.7071067811865476
    res = jnp.where(y > 0.0, _erf_pos(jnp.maximum(y, 0.0)), 0.0)
    out_ref[:] = res


@jax.jit
def _run(x, train_points):
    pad = jnp.full((_NP - _N, _D), _PAD, dtype=jnp.float32)
    tt = jnp.concatenate([train_points, pad], axis=0).T
    tt_hi = tt.astype(jnp.bfloat16)
    r1 = tt - tt_hi.astype(jnp.float32)
    tt_mid = r1.astype(jnp.bfloat16)
    tt_lo = (r1 - tt_mid.astype(jnp.float32)).astype(jnp.bfloat16)
    out = pl.pallas_call(
        _loop_kernel,
        out_shape=jax.ShapeDtypeStruct((_Q, 1), jnp.float32),
        compiler_params=pltpu.CompilerParams(
            vmem_limit_bytes=100 * 1024 * 1024),
    )(x, tt, tt_hi, tt_mid, tt_lo)
    return out.reshape((_Q,))


def kernel(X, train_points):
    return _run(X, train_points)


# speedup vs baseline: 3.7343x; 1.7436x over previous
import jax
import jax.numpy as jnp
from jax.experimental import pallas as pl
from jax.experimental.pallas import tpu as pltpu

_K = 8
_LAMBDA = 3.0
_N = 100000
_NP = 102400
_D = 16
_Q = 8
_PAD = 1.0e6
_MASKV = 1.0e30


def _min_lanes(a):
    w = a.shape[1]
    while w > 3200:
        w //= 2
        a = jnp.minimum(a[:, :w], a[:, w:])
    return jnp.min(a, axis=1, keepdims=True)


def _topk8(d2, iota, iota8):
    def body(j, carry):
        d2c, s, inds = carry
        m = _min_lanes(d2c)
        cand = jnp.where(d2c == m, iota, _NP)
        idx = _min_lanes(cand)
        d2c = jnp.where(iota == idx, _MASKV, d2c)
        s = s + m
        inds = jnp.where(iota8 == j, jnp.broadcast_to(idx, (_Q, _K)), inds)
        return (d2c, s, inds)

    init = (d2, jnp.zeros((_Q, 1), jnp.float32), jnp.zeros((_Q, _K), jnp.int32))
    _, s, inds = jax.lax.fori_loop(0, _K, body, init)
    return s, inds


_CW = 1024
_NC = _NP // _CW
_RW = 128


def _topk8_hier(d2, iota1024, iota128, iota8):
    mk = lambda v, dt: jnp.full((_Q, _RW), v, dtype=dt)
    r1, r2, r3 = mk(_MASKV, jnp.float32), mk(_MASKV, jnp.float32), mk(_MASKV, jnp.float32)
    al1, al2 = mk(0, jnp.int32), mk(0, jnp.int32)
    for c in range(_NC):
        slab = d2[:, c * _CW:(c + 1) * _CW]
        m1 = jnp.min(slab, axis=1, keepdims=True)
        l1 = jnp.min(jnp.where(slab == m1, iota1024, _CW), axis=1, keepdims=True)
        slab2 = jnp.where(iota1024 == l1, _MASKV, slab)
        m2 = jnp.min(slab2, axis=1, keepdims=True)
        l2 = jnp.min(jnp.where(slab2 == m2, iota1024, _CW), axis=1, keepdims=True)
        m3 = jnp.min(jnp.where(iota1024 == l2, _MASKV, slab2), axis=1, keepdims=True)
        sel = iota128 == c
        r1 = jnp.where(sel, m1, r1)
        r2 = jnp.where(sel, m2, r2)
        r3 = jnp.where(sel, m3, r3)
        al1 = jnp.where(sel, l1, al1)
        al2 = jnp.where(sel, l2, al2)

    s = jnp.zeros((_Q, 1), jnp.float32)
    inds = jnp.zeros((_Q, _K), jnp.int32)
    pops = jnp.zeros((_Q, _RW), jnp.int32)
    bad = jnp.zeros((_Q, 1), jnp.int32)
    for j in range(_K):
        m = jnp.min(r1, axis=1, keepdims=True)
        cs = jnp.min(jnp.where(r1 == m, iota128, _RW), axis=1, keepdims=True)
        sel = iota128 == cs
        lane = jnp.min(jnp.where(sel, jnp.where(pops == 0, al1, al2), _CW),
                       axis=1, keepdims=True)
        idx = cs * _CW + lane
        s = s + m
        inds = jnp.where(iota8 == j, jnp.broadcast_to(idx, (_Q, _K)), inds)
        bad = bad + jnp.max(jnp.where(sel & (pops >= 2), 1, 0),
                            axis=1, keepdims=True)
        repl = jnp.where(pops == 0, r2, jnp.where(pops == 1, r3, _MASKV))
        r1 = jnp.where(sel, repl, r1)
        pops = pops + sel.astype(jnp.int32)
    return s, inds, bad


def _pdist_from_sum(s):
    s = jnp.maximum(s, 0.0)
    return _LAMBDA * jnp.sqrt(s * (1.0 / _K))


def _erf_pos(x):
    t = 1.0 / (1.0 + 0.3275911 * x)
    poly = ((((1.061405429 * t - 1.453152027) * t + 1.421413741) * t
             - 0.284496736) * t + 0.254829592) * t
    return 1.0 - poly * jnp.exp(-(x * x))


def _loop_kernel(x_ref, tt_ref, tt_hi_ref, tt_mid_ref, tt_lo_ref, out_ref):
    tt = tt_ref[:]
    tn = jnp.sum(tt * tt, axis=0, keepdims=True)
    iota = jax.lax.broadcasted_iota(jnp.int32, (_Q, _NP), 1)
    iota8 = jax.lax.broadcasted_iota(jnp.int32, (_Q, _K), 1)
    iota1024 = jax.lax.broadcasted_iota(jnp.int32, (_Q, _CW), 1)
    iota128 = jax.lax.broadcasted_iota(jnp.int32, (_Q, _RW), 1)

    def topk(d2):
        s, inds, bad = _topk8_hier(d2, iota1024, iota128, iota8)
        return jax.lax.cond(jnp.max(bad) > 0,
                            lambda: _topk8(d2, iota, iota8),
                            lambda: (s, inds))

    x = x_ref[:]
    qt = jax.lax.dot_general(x, tt, (((1,), (0,)), ((), ())),
                             preferred_element_type=jnp.float32,
                             precision=jax.lax.Precision.HIGHEST)
    qn = jnp.sum(x * x, axis=1, keepdims=True)
    d2 = tn - 2.0 * qt + qn
    s1, inds1 = topk(d2)
    pd = _pdist_from_sum(s1)

    eye = (jax.lax.broadcasted_iota(jnp.int32, (_Q, _Q), 0)
           == jax.lax.broadcasted_iota(jnp.int32, (_Q, _Q), 1)).astype(jnp.float32)

    def group_body(g, nf_sum):
        col = jnp.min(jnp.where(iota8 == g, inds1, _NP), axis=1, keepdims=True)
        onehot = (iota == col).astype(jnp.bfloat16)
        q2t = (jax.lax.dot_general(tt_hi_ref[:], onehot, (((1,), (1,)), ((), ())),
                                   preferred_element_type=jnp.float32)
               + jax.lax.dot_general(tt_mid_ref[:], onehot, (((1,), (1,)), ((), ())),
                                     preferred_element_type=jnp.float32)
               + jax.lax.dot_general(tt_lo_ref[:], onehot, (((1,), (1,)), ((), ())),
                                     preferred_element_type=jnp.float32))
        qt2 = jax.lax.dot_general(q2t, tt, (((0,), (0,)), ((), ())),
                                  preferred_element_type=jnp.float32,
                             precision=jax.lax.Precision.HIGHEST)
        gram = jax.lax.dot_general(q2t, q2t, (((0,), (0,)), ((), ())),
                                   preferred_element_type=jnp.float32,
                             precision=jax.lax.Precision.HIGHEST)
        qn2 = jnp.sum(gram * eye, axis=1, keepdims=True)
        d2b = tn - 2.0 * qt2 + qn2
        s2, _ = topk(d2b)
        return nf_sum + _pdist_from_sum(s2)

    nf_sum = jax.lax.fori_loop(0, _K, group_body,
                               jnp.zeros((_Q, 1), jnp.float32))

    nf = nf_sum * (1.0 / _K)
    lof = pd / nf - 1.0
    y = lof * 0.7071067811865476
    res = jnp.where(y > 0.0, _erf_pos(jnp.maximum(y, 0.0)), 0.0)
    out_ref[:] = res


@jax.jit
def _run(x, train_points):
    pad = jnp.full((_NP - _N, _D), _PAD, dtype=jnp.float32)
    tt = jnp.concatenate([train_points, pad], axis=0).T
    tt_hi = tt.astype(jnp.bfloat16)
    r1 = tt - tt_hi.astype(jnp.float32)
    tt_mid = r1.astype(jnp.bfloat16)
    tt_lo = (r1 - tt_mid.astype(jnp.float32)).astype(jnp.bfloat16)
    out = pl.pallas_call(
        _loop_kernel,
        out_shape=jax.ShapeDtypeStruct((_Q, 1), jnp.float32),
        compiler_params=pltpu.CompilerParams(
            vmem_limit_bytes=100 * 1024 * 1024),
    )(x, tt, tt_hi, tt_mid, tt_lo)
    return out.reshape((_Q,))


def kernel(X, train_points):
    return _run(X, train_points)


# speedup vs baseline: 5.1654x; 1.3832x over previous
import jax
import jax.numpy as jnp
from jax.experimental import pallas as pl
from jax.experimental.pallas import tpu as pltpu

_K = 8
_LAMBDA = 3.0
_N = 100000
_NP = 102400
_D = 16
_Q = 8
_PAD = 1.0e6
_MASKV = 1.0e30


def _min_lanes(a):
    w = a.shape[1]
    while w > 3200:
        w //= 2
        a = jnp.minimum(a[:, :w], a[:, w:])
    return jnp.min(a, axis=1, keepdims=True)


def _topk8(d2, iota, iota8):
    def body(j, carry):
        d2c, s, inds = carry
        m = _min_lanes(d2c)
        cand = jnp.where(d2c == m, iota, _NP)
        idx = _min_lanes(cand)
        d2c = jnp.where(iota == idx, _MASKV, d2c)
        s = s + m
        inds = jnp.where(iota8 == j, jnp.broadcast_to(idx, (_Q, _K)), inds)
        return (d2c, s, inds)

    init = (d2, jnp.zeros((_Q, 1), jnp.float32), jnp.zeros((_Q, _K), jnp.int32))
    _, s, inds = jax.lax.fori_loop(0, _K, body, init)
    return s, inds


_CW = 1024
_NC = _NP // _CW
_RW = 128


def _topk8_hier(d2, iota1024, iota128, iotak):
    rows = d2.shape[0]
    mk = lambda v, dt: jnp.full((rows, _RW), v, dtype=dt)
    r1, r2, r3, r4 = (mk(_MASKV, jnp.float32) for _ in range(4))
    al1, al2, al3 = (mk(0, jnp.int32) for _ in range(3))
    for c in range(_NC):
        slab = d2[:, c * _CW:(c + 1) * _CW]
        m1 = jnp.min(slab, axis=1, keepdims=True)
        l1 = jnp.min(jnp.where(slab == m1, iota1024, _CW), axis=1, keepdims=True)
        slab2 = jnp.where(iota1024 == l1, _MASKV, slab)
        m2 = jnp.min(slab2, axis=1, keepdims=True)
        l2 = jnp.min(jnp.where(slab2 == m2, iota1024, _CW), axis=1, keepdims=True)
        slab3 = jnp.where(iota1024 == l2, _MASKV, slab2)
        m3 = jnp.min(slab3, axis=1, keepdims=True)
        l3 = jnp.min(jnp.where(slab3 == m3, iota1024, _CW), axis=1, keepdims=True)
        m4 = jnp.min(jnp.where(iota1024 == l3, _MASKV, slab3), axis=1, keepdims=True)
        sel = iota128 == c
        r1 = jnp.where(sel, m1, r1)
        r2 = jnp.where(sel, m2, r2)
        r3 = jnp.where(sel, m3, r3)
        r4 = jnp.where(sel, m4, r4)
        al1 = jnp.where(sel, l1, al1)
        al2 = jnp.where(sel, l2, al2)
        al3 = jnp.where(sel, l3, al3)

    s = jnp.zeros((rows, 1), jnp.float32)
    inds = jnp.zeros((rows, _K), jnp.int32)
    pops = jnp.zeros((rows, _RW), jnp.int32)
    bad = jnp.zeros((rows, 1), jnp.int32)
    for j in range(_K):
        m = jnp.min(r1, axis=1, keepdims=True)
        cs = jnp.min(jnp.where(r1 == m, iota128, _RW), axis=1, keepdims=True)
        sel = iota128 == cs
        alcur = jnp.where(pops == 0, al1, jnp.where(pops == 1, al2, al3))
        lane = jnp.min(jnp.where(sel, alcur, _CW), axis=1, keepdims=True)
        idx = cs * _CW + lane
        s = s + m
        inds = jnp.where(iotak == j, jnp.broadcast_to(idx, (rows, _K)), inds)
        bad = bad + jnp.max(jnp.where(sel & (pops >= 3), 1, 0),
                            axis=1, keepdims=True)
        repl = jnp.where(pops == 0, r2,
                         jnp.where(pops == 1, r3,
                                   jnp.where(pops == 2, r4, _MASKV)))
        r1 = jnp.where(sel, repl, r1)
        pops = pops + sel.astype(jnp.int32)
    return s, inds, bad


def _pdist_from_sum(s):
    s = jnp.maximum(s, 0.0)
    return _LAMBDA * jnp.sqrt(s * (1.0 / _K))


def _erf_pos(x):
    t = 1.0 / (1.0 + 0.3275911 * x)
    poly = ((((1.061405429 * t - 1.453152027) * t + 1.421413741) * t
             - 0.284496736) * t + 0.254829592) * t
    return 1.0 - poly * jnp.exp(-(x * x))


def _loop_kernel(x_ref, tt_ref, tt_hi_ref, tt_mid_ref, tt_lo_ref, out_ref):
    tt = tt_ref[:]
    tn = jnp.sum(tt * tt, axis=0, keepdims=True)
    iota = jax.lax.broadcasted_iota(jnp.int32, (_Q, _NP), 1)
    iota8 = jax.lax.broadcasted_iota(jnp.int32, (_Q, _K), 1)
    iota1024 = jax.lax.broadcasted_iota(jnp.int32, (_Q, _CW), 1)
    iota128 = jax.lax.broadcasted_iota(jnp.int32, (_Q, _RW), 1)

    iota1024_16 = jax.lax.broadcasted_iota(jnp.int32, (16, _CW), 1)
    iota128_16 = jax.lax.broadcasted_iota(jnp.int32, (16, _RW), 1)
    iotak16 = jax.lax.broadcasted_iota(jnp.int32, (16, _K), 1)

    def topk(d2):
        s, inds, bad = _topk8_hier(d2, iota1024, iota128, iota8)
        return jax.lax.cond(jnp.max(bad) > 0,
                            lambda: _topk8(d2, iota, iota8),
                            lambda: (s, inds))

    x = x_ref[:]
    qt = jax.lax.dot_general(x, tt, (((1,), (0,)), ((), ())),
                             preferred_element_type=jnp.float32,
                             precision=jax.lax.Precision.HIGHEST)
    qn = jnp.sum(x * x, axis=1, keepdims=True)
    d2 = tn - 2.0 * qt + qn
    s1, inds1 = topk(d2)
    pd = _pdist_from_sum(s1)

    eye16 = (jax.lax.broadcasted_iota(jnp.int32, (16, 16), 0)
             == jax.lax.broadcasted_iota(jnp.int32, (16, 16), 1)).astype(jnp.float32)
    def half_body(h, nf_sum):
        oh_blocks = []
        for j in range(2):
            jj = 2 * h + j
            col = jnp.min(jnp.where(iota8 == jj, inds1, _NP), axis=1, keepdims=True)
            oh_blocks.append((iota == col).astype(jnp.bfloat16))
        oh16 = jnp.concatenate(oh_blocks, axis=0)
        q2t = (jax.lax.dot_general(tt_hi_ref[:], oh16, (((1,), (1,)), ((), ())),
                                   preferred_element_type=jnp.float32)
               + jax.lax.dot_general(tt_mid_ref[:], oh16, (((1,), (1,)), ((), ())),
                                     preferred_element_type=jnp.float32)
               + jax.lax.dot_general(tt_lo_ref[:], oh16, (((1,), (1,)), ((), ())),
                                     preferred_element_type=jnp.float32))
        qt2 = jax.lax.dot_general(q2t, tt, (((0,), (0,)), ((), ())),
                                  preferred_element_type=jnp.float32,
                                  precision=jax.lax.Precision.HIGHEST)
        gram = jax.lax.dot_general(q2t, q2t, (((0,), (0,)), ((), ())),
                                   preferred_element_type=jnp.float32,
                                   precision=jax.lax.Precision.HIGHEST)
        qn2 = jnp.sum(gram * eye16, axis=1, keepdims=True)
        d2b = tn - 2.0 * qt2 + qn2

        s16h, _, bad16 = _topk8_hier(d2b, iota1024_16, iota128_16, iotak16)

        def _fallback16(d2b=d2b):
            parts = [_topk8(d2b[j * _Q:(j + 1) * _Q], iota, iota8)[0]
                     for j in range(2)]
            return jnp.concatenate(parts, axis=0)

        s16 = jax.lax.cond(jnp.max(bad16) > 0, _fallback16, lambda s=s16h: s)
        pdg = _pdist_from_sum(s16)
        for j in range(2):
            nf_sum = nf_sum + pdg[j * _Q:(j + 1) * _Q]
        return nf_sum

    nf_sum = jax.lax.fori_loop(0, 4, half_body,
                               jnp.zeros((_Q, 1), jnp.float32))

    nf = nf_sum * (1.0 / _K)
    lof = pd / nf - 1.0
    y = lof * 0.7071067811865476
    res = jnp.where(y > 0.0, _erf_pos(jnp.maximum(y, 0.0)), 0.0)
    out_ref[:] = res


@jax.jit
def _run(x, train_points):
    pad = jnp.full((_NP - _N, _D), _PAD, dtype=jnp.float32)
    tt = jnp.concatenate([train_points, pad], axis=0).T
    tt_hi = tt.astype(jnp.bfloat16)
    r1 = tt - tt_hi.astype(jnp.float32)
    tt_mid = r1.astype(jnp.bfloat16)
    tt_lo = (r1 - tt_mid.astype(jnp.float32)).astype(jnp.bfloat16)
    out = pl.pallas_call(
        _loop_kernel,
        out_shape=jax.ShapeDtypeStruct((_Q, 1), jnp.float32),
        compiler_params=pltpu.CompilerParams(
            vmem_limit_bytes=67000000),
    )(x, tt, tt_hi, tt_mid, tt_lo)
    return out.reshape((_Q,))


def kernel(X, train_points):
    return _run(X, train_points)
